# Initial kernel scaffold; baseline (speedup 1.0000x reference)
import jax
import jax.numpy as jnp
from jax.experimental import pallas as pl


def kernel(x, node_table, pre_table):
    raise NotImplementedError("write your pallas kernel here")


# speedup vs baseline: 2.1313x; 2.1313x over previous
import functools

import jax
import jax.numpy as jnp
from jax import lax
from jax.experimental import pallas as pl
from jax.experimental.pallas import tpu as pltpu
from jax.experimental.pallas import tpu_sc as plsc

_B = 1024
_LEN = 200
_D = 64
_NC, _NS = 2, 16
_NW = _NC * _NS
_SPW = _B // _NW
_H = 104
_NIDS = _LEN - 2
_ROWS = 212
_NBUF = 4
_ITER = _SPW // _NBUF


def _sc_body(nodes, table, pre, out, idx_v, rows_v, sep_v, gsem, osem):
    c = lax.axis_index("c")
    s = lax.axis_index("s")
    wid = s * _NC + c
    base = wid * _SPW

    pltpu.sync_copy(nodes.at[pl.ds(base, _SPW)], idx_v)
    pltpu.sync_copy(pre.at[1], sep_v)
    for b in range(_NBUF):
        pltpu.sync_copy(pre.at[0], rows_v.at[b, 0])

    def gather_descs(seq_local, b):
        return (
            pltpu.make_async_copy(table.at[idx_v.at[seq_local, 0]],
                                  rows_v.at[b, pl.ds(1, _H)], gsem.at[b]),
            pltpu.make_async_copy(table.at[idx_v.at[seq_local, 1]],
                                  rows_v.at[b, pl.ds(1 + _H, _H)], gsem.at[b]),
        )

    def out_desc(seq, b):
        return pltpu.make_async_copy(rows_v.at[b, pl.ds(0, _LEN)],
                                     out.at[seq], osem.at[b])

    for b in range(_NBUF):
        for d in gather_descs(b, b):
            d.start()

    def step(i, carry):
        r0 = i * _NBUF
        for b in range(_NBUF):
            sl = r0 + b
            for d in gather_descs(sl, b):
                d.wait()
            for j in range(_D // 16):
                rows_v[b, _LEN - 1, pl.ds(j * 16, 16)] = sep_v[pl.ds(j * 16, 16)]
            out_desc(base + sl, b).start()

        @pl.when(i + 1 < _ITER)
        def _():
            for b in range(_NBUF):
                nsl = r0 + _NBUF + b
                out_desc(base + nsl - _NBUF, b).wait()
                for d in gather_descs(nsl, b):
                    d.start()

        return carry

    lax.fori_loop(0, _ITER, step, 0)

    for b in range(_NBUF):
        out_desc(base + (_ITER - 1) * _NBUF + b, b).wait()


_gather_call = functools.partial(
    pl.kernel,
    out_type=jax.ShapeDtypeStruct((_B, _LEN, _D), jnp.float32),
    mesh=plsc.VectorSubcoreMesh(core_axis_name="c", subcore_axis_name="s"),
    compiler_params=pltpu.CompilerParams(use_tc_tiling_on_sc=False),
    scratch_types=[
        pltpu.VMEM((_SPW, 2, _H), jnp.int32),
        pltpu.VMEM((_NBUF, _ROWS, _D), jnp.float32),
        pltpu.VMEM((_D,), jnp.float32),
        pltpu.SemaphoreType.DMA((_NBUF,)),
        pltpu.SemaphoreType.DMA((_NBUF,)),
    ],
)(_sc_body)


@jax.jit
def kernel(x, node_table, pre_table):
    nodes = x[:, 1:_LEN - 1].astype(jnp.int32)
    nodes = jnp.pad(nodes, ((0, 0), (0, 2 * _H - _NIDS)))
    nodes = nodes.reshape(_B, 2, _H)
    return _gather_call(nodes, node_table, pre_table)


# speedup vs baseline: 3.9797x; 1.8673x over previous
import functools

import jax
import jax.numpy as jnp
from jax import lax
from jax.experimental import pallas as pl
from jax.experimental.pallas import tpu as pltpu
from jax.experimental.pallas import tpu_sc as plsc

_B = 1024
_LEN = 200
_D = 64
_NC, _NS = 2, 16
_NW = _NC * _NS
_SPW = _B // _NW
_S = 2
_NBUF = 4
_ITER = _SPW // (_S * _NBUF)


def _sc_body(x, table, pre, out, idx_v, rows_v, cls_v, sep_v, gsem, osem):
    c = lax.axis_index("c")
    s = lax.axis_index("s")
    wid = s * _NC + c
    base = wid * _SPW

    pltpu.sync_copy(x.at[pl.ds(base, _SPW)], idx_v)
    pltpu.sync_copy(pre.at[0], cls_v)
    pltpu.sync_copy(pre.at[1], sep_v)

    def gather_descs(slot_seq0, b):
        return tuple(
            pltpu.make_async_copy(table.at[idx_v.at[slot_seq0 + q, h]],
                                  rows_v.at[b, q, h], gsem.at[b])
            for q in range(_S) for h in range(2)
        )

    def out_desc(seq0, b):
        return pltpu.make_async_copy(rows_v.at[b],
                                     out.at[pl.ds(seq0, _S)], osem.at[b])

    def stamp(b):
        for q in range(_S):
            for j in range(_D // 16):
                lanes = pl.ds(j * 16, 16)
                rows_v[b, q, 0, 0, lanes] = cls_v[lanes]
                rows_v[b, q, 1, 99, lanes] = sep_v[lanes]

    for b in range(_NBUF):
        for d in gather_descs(b * _S, b):
            d.start()

    def step(i, carry):
        r0 = i * _NBUF * _S
        for b in range(_NBUF):
            sl = r0 + b * _S
            for d in gather_descs(sl, b):
                d.wait()
            stamp(b)
            out_desc(base + sl, b).start()

        @pl.when(i + 1 < _ITER)
        def _():
            for b in range(_NBUF):
                nsl = r0 + _NBUF * _S + b * _S
                out_desc(base + nsl - _NBUF * _S, b).wait()
                for d in gather_descs(nsl, b):
                    d.start()

        return carry

    lax.fori_loop(0, _ITER, step, 0)

    for b in range(_NBUF):
        out_desc(base + (_ITER - 1) * _NBUF * _S + b * _S, b).wait()


_gather_call = functools.partial(
    pl.kernel,
    out_type=jax.ShapeDtypeStruct((_B, 2, _LEN // 2, _D), jnp.float32),
    mesh=plsc.VectorSubcoreMesh(core_axis_name="c", subcore_axis_name="s"),
    compiler_params=pltpu.CompilerParams(use_tc_tiling_on_sc=False),
    scratch_types=[
        pltpu.VMEM((_SPW, 2, _LEN // 2), jnp.int32),
        pltpu.VMEM((_NBUF, _S, 2, _LEN // 2, _D), jnp.float32),
        pltpu.VMEM((_D,), jnp.float32),
        pltpu.VMEM((_D,), jnp.float32),
        pltpu.SemaphoreType.DMA((_NBUF,)),
        pltpu.SemaphoreType.DMA((_NBUF,)),
    ],
)(_sc_body)


@jax.jit
def kernel(x, node_table, pre_table):
    x3 = x.astype(jnp.int32).reshape(_B, 2, _LEN // 2)
    emb = _gather_call(x3, node_table, pre_table)
    return emb.reshape(_B, _LEN, _D)


# speedup vs baseline: 4.3604x; 1.0957x over previous
import functools

import jax
import jax.numpy as jnp
from jax import lax
from jax.experimental import pallas as pl
from jax.experimental.pallas import tpu as pltpu
from jax.experimental.pallas import tpu_sc as plsc

_B = 1024
_LEN = 200
_D = 64
_H = _LEN // 2
_NC, _NS = 2, 16
_NW = _NC * _NS
_SPW = _B // _NW
_S = 2
_NBUF = 4
_ITER = _SPW // (_S * _NBUF)


def _sc_body(x, table, pre, out, idx_v, rows_v, cls_v, sep_v, gsem, osem):
    c = lax.axis_index("c")
    s = lax.axis_index("s")
    wid = s * _NC + c
    base = wid * _SPW

    pltpu.sync_copy(x.at[pl.ds(base, _SPW)], idx_v)
    pltpu.sync_copy(pre.at[0], cls_v)
    pltpu.sync_copy(pre.at[1], sep_v)

    def gather_descs(slot_seq0, b):
        return tuple(
            pltpu.make_async_copy(
                table.at[idx_v.at[slot_seq0 + q, pl.ds(off, ln)]],
                rows_v.at[b, q, pl.ds(off, ln)],
                gsem.at[b])
            for q in range(_S) for off, ln in ((0, 104), (104, 96))
        )

    def out_desc(seq0, b):
        return pltpu.make_async_copy(rows_v.at[b],
                                     out.at[pl.ds(seq0, _S)], osem.at[b])

    def stamp(b):
        for q in range(_S):
            for j in range(_D // 16):
                lanes = pl.ds(j * 16, 16)
                rows_v[b, q, 0, lanes] = cls_v[lanes]
                rows_v[b, q, _LEN - 1, lanes] = sep_v[lanes]

    for b in range(_NBUF):
        for d in gather_descs(b * _S, b):
            d.start()

    def step(i, carry):
        r0 = i * _NBUF * _S
        for b in range(_NBUF):
            sl = r0 + b * _S
            for d in gather_descs(sl, b):
                d.wait()
            stamp(b)
            out_desc(base + sl, b).start()

        @pl.when(i + 1 < _ITER)
        def _():
            for b in range(_NBUF):
                nsl = r0 + _NBUF * _S + b * _S
                out_desc(base + nsl - _NBUF * _S, b).wait()
                for d in gather_descs(nsl, b):
                    d.start()

        return carry

    lax.fori_loop(0, _ITER, step, 0)

    for b in range(_NBUF):
        out_desc(base + (_ITER - 1) * _NBUF * _S + b * _S, b).wait()


_gather_call = functools.partial(
    pl.kernel,
    out_type=jax.ShapeDtypeStruct((_B, _LEN, _D), jnp.float32),
    mesh=plsc.VectorSubcoreMesh(core_axis_name="c", subcore_axis_name="s"),
    compiler_params=pltpu.CompilerParams(use_tc_tiling_on_sc=False),
    scratch_types=[
        pltpu.VMEM((_SPW, _LEN), jnp.int32),
        pltpu.VMEM((_NBUF, _S, _LEN, _D), jnp.float32),
        pltpu.VMEM((_D,), jnp.float32),
        pltpu.VMEM((_D,), jnp.float32),
        pltpu.SemaphoreType.DMA((_NBUF,)),
        pltpu.SemaphoreType.DMA((_NBUF,)),
    ],
)(_sc_body)


@jax.jit
def kernel(x, node_table, pre_table):
    return _gather_call(x.astype(jnp.int32), node_table, pre_table)
